# baseline (speedup 1.0000x reference)
import jax
import jax.numpy as jnp
from jax.experimental import pallas as pl

B = 1024
N1, N2 = 25, 10
DIN = 128
H0, H1 = 256, 128
NODES = 1 + N1 + N1 * N2
BB = 32


def _leaky(x):
    return jnp.where(x >= 0, x, x * 0.01)


def _tower(f, w1a, w1b, b1, w2a, w2b, b2):
    def node(i):
        return f[:, DIN * i:DIN * (i + 1)]

    h1 = [node(1 + j) for j in range(N1)]
    m0 = sum(h1[1:], h1[0]) * (1.0 / N1)
    means = [m0]
    for j in range(N1):
        base = 1 + N1 + N2 * j
        s = node(base)
        for k in range(1, N2):
            s = s + node(base + k)
        means.append(s * (1.0 / N2))

    xh = jnp.concatenate([node(0)] + h1, axis=0)
    xn = jnp.concatenate(means, axis=0)

    l1 = _leaky(
        jnp.dot(xh, w1a, preferred_element_type=jnp.float32)
        + jnp.dot(xn, w1b, preferred_element_type=jnp.float32)
        + b1
    )

    h0n = l1[0:BB]
    neigh = l1[BB:2 * BB]
    for j in range(2, N1 + 1):
        neigh = neigh + l1[j * BB:(j + 1) * BB]
    neigh = neigh * (1.0 / N1)

    h0f = _leaky(
        jnp.dot(h0n, w2a, preferred_element_type=jnp.float32)
        + jnp.dot(neigh, w2b, preferred_element_type=jnp.float32)
        + b2
    )
    return _leaky(h0f)


def _fused_kernel(uf_ref, if_ref, w1ua_ref, w1ub_ref, b1u_ref, w2ua_ref,
                  w2ub_ref, b2u_ref, w1ia_ref, w1ib_ref, b1i_ref, w2ia_ref,
                  w2ib_ref, b2i_ref, wl_ref, bl_ref, out_ref):
    uh = _tower(uf_ref[...], w1ua_ref[...], w1ub_ref[...], b1u_ref[...],
                w2ua_ref[...], w2ub_ref[...], b2u_ref[...])
    ih = _tower(if_ref[...], w1ia_ref[...], w1ib_ref[...], b1i_ref[...],
                w2ia_ref[...], w2ib_ref[...], b2i_ref[...])
    p = uh * ih
    z = jnp.dot(p, wl_ref[...], preferred_element_type=jnp.float32) + bl_ref[...]
    out_ref[...] = jax.nn.sigmoid(z)


def kernel(sampling_user_feat, sampling_item_feat, W1_u, b1_u, W2_u, b2_u,
           W1_i, b1_i, W2_i, b2_i, W_lin, b_lin):
    uf = sampling_user_feat.reshape(B, NODES * DIN)
    itf = sampling_item_feat.reshape(B, NODES * DIN)
    w1ua, w1ub = W1_u[:DIN], W1_u[DIN:]
    w2ua, w2ub = W2_u[:H0], W2_u[H0:]
    w1ia, w1ib = W1_i[:DIN], W1_i[DIN:]
    w2ia, w2ib = W2_i[:H0], W2_i[H0:]
    b1u = b1_u.reshape(1, H0)
    b2u = b2_u.reshape(1, H1)
    b1i = b1_i.reshape(1, H0)
    b2i = b2_i.reshape(1, H1)
    wl = jnp.zeros((H1, 128), jnp.float32).at[:, :2].set(W_lin)
    bl = jnp.zeros((1, 128), jnp.float32).at[:, :2].set(b_lin)

    grid = B // BB
    feat_spec = pl.BlockSpec((BB, NODES * DIN), lambda i: (i, 0))

    def wspec(shape):
        return pl.BlockSpec(shape, lambda i: tuple(0 for _ in shape))

    out = pl.pallas_call(
        _fused_kernel,
        grid=(grid,),
        in_specs=[
            feat_spec, feat_spec,
            wspec((DIN, H0)), wspec((DIN, H0)), wspec((1, H0)),
            wspec((H0, H1)), wspec((H0, H1)), wspec((1, H1)),
            wspec((DIN, H0)), wspec((DIN, H0)), wspec((1, H0)),
            wspec((H0, H1)), wspec((H0, H1)), wspec((1, H1)),
            wspec((H1, 128)), wspec((1, 128)),
        ],
        out_specs=pl.BlockSpec((BB, 128), lambda i: (i, 0)),
        out_shape=jax.ShapeDtypeStruct((B, 128), jnp.float32),
    )(uf, itf,
      w1ua, w1ub, b1u, w2ua, w2ub, b2u,
      w1ia, w1ib, b1i, w2ia, w2ib, b2i, wl, bl)
    return out[:, :2]


# speedup vs baseline: 1.4168x; 1.4168x over previous
import jax
import jax.numpy as jnp
from jax.experimental import pallas as pl

B = 1024
N1, N2 = 25, 10
DIN = 128
H0, H1 = 256, 128
NODES = 1 + N1 + N1 * N2
BB = 64
PAD = 32


def _leaky(x):
    return jnp.where(x >= 0, x, x * 0.01)


def _tower(f, w1a, w1b, b1, w2a, w2b, b2):
    h32 = f[:, 0:PAD, :]
    parts = [jnp.mean(f[:, 1:1 + N1, :], axis=1, keepdims=True)]
    for j in range(N1):
        lo = 1 + N1 + N2 * j
        parts.append(jnp.mean(f[:, lo:lo + N2, :], axis=1, keepdims=True))
    parts.append(jnp.zeros((BB, PAD - 1 - N1, DIN), jnp.float32))
    n32 = jnp.concatenate(parts, axis=1)

    hf = h32.reshape(BB * PAD, DIN)
    nf = n32.reshape(BB * PAD, DIN)
    l1 = _leaky(
        jnp.dot(hf, w1a, preferred_element_type=jnp.float32)
        + jnp.dot(nf, w1b, preferred_element_type=jnp.float32)
        + b1
    ).reshape(BB, PAD, H0)

    h0n = l1[:, 0, :]
    neigh = jnp.mean(l1[:, 1:1 + N1, :], axis=1)
    h0f = _leaky(
        jnp.dot(h0n, w2a, preferred_element_type=jnp.float32)
        + jnp.dot(neigh, w2b, preferred_element_type=jnp.float32)
        + b2
    )
    return _leaky(h0f)


def _fused_kernel(uf_ref, if_ref, w1ua_ref, w1ub_ref, b1u_ref, w2ua_ref,
                  w2ub_ref, b2u_ref, w1ia_ref, w1ib_ref, b1i_ref, w2ia_ref,
                  w2ib_ref, b2i_ref, wl_ref, bl_ref, out_ref):
    uh = _tower(uf_ref[...], w1ua_ref[...], w1ub_ref[...], b1u_ref[...],
                w2ua_ref[...], w2ub_ref[...], b2u_ref[...])
    ih = _tower(if_ref[...], w1ia_ref[...], w1ib_ref[...], b1i_ref[...],
                w2ia_ref[...], w2ib_ref[...], b2i_ref[...])
    p = uh * ih
    z = jnp.dot(p, wl_ref[...], preferred_element_type=jnp.float32) + bl_ref[...]
    out_ref[...] = jax.nn.sigmoid(z)


def kernel(sampling_user_feat, sampling_item_feat, W1_u, b1_u, W2_u, b2_u,
           W1_i, b1_i, W2_i, b2_i, W_lin, b_lin):
    w1ua, w1ub = W1_u[:DIN], W1_u[DIN:]
    w2ua, w2ub = W2_u[:H0], W2_u[H0:]
    w1ia, w1ib = W1_i[:DIN], W1_i[DIN:]
    w2ia, w2ib = W2_i[:H0], W2_i[H0:]
    b1u = b1_u.reshape(1, H0)
    b2u = b2_u.reshape(1, H1)
    b1i = b1_i.reshape(1, H0)
    b2i = b2_i.reshape(1, H1)
    wl = jnp.zeros((H1, 128), jnp.float32).at[:, :2].set(W_lin)
    bl = jnp.zeros((1, 128), jnp.float32).at[:, :2].set(b_lin)

    grid = B // BB
    feat_spec = pl.BlockSpec((BB, NODES, DIN), lambda i: (i, 0, 0))

    def wspec(shape):
        return pl.BlockSpec(shape, lambda i: tuple(0 for _ in shape))

    out = pl.pallas_call(
        _fused_kernel,
        grid=(grid,),
        in_specs=[
            feat_spec, feat_spec,
            wspec((DIN, H0)), wspec((DIN, H0)), wspec((1, H0)),
            wspec((H0, H1)), wspec((H0, H1)), wspec((1, H1)),
            wspec((DIN, H0)), wspec((DIN, H0)), wspec((1, H0)),
            wspec((H0, H1)), wspec((H0, H1)), wspec((1, H1)),
            wspec((H1, 128)), wspec((1, 128)),
        ],
        out_specs=pl.BlockSpec((BB, 128), lambda i: (i, 0)),
        out_shape=jax.ShapeDtypeStruct((B, 128), jnp.float32),
    )(sampling_user_feat, sampling_item_feat,
      w1ua, w1ub, b1u, w2ua, w2ub, b2u,
      w1ia, w1ib, b1i, w2ia, w2ib, b2i, wl, bl)
    return out[:, :2]
